# Initial kernel scaffold; baseline (speedup 1.0000x reference)
import jax
import jax.numpy as jnp
from jax.experimental import pallas as pl


def kernel(path_segment_ids, path_segment_feats, edge_feats, emb_table, W_emb, b_emb, W_inner, b_inner, W_inter, b_inter, W_fwd1, b_fwd1, W_fwd2, b_fwd2, W_due, W_act):
    raise NotImplementedError("write your pallas kernel here")


# speedup vs baseline: 1.5336x; 1.5336x over previous
import functools

import jax
import jax.numpy as jnp
from jax import lax
from jax.experimental import pallas as pl
from jax.experimental.pallas import tpu as pltpu
from jax.experimental.pallas import tpu_sc as plsc

EDGE_NUM = 10000
D_FEAT = 16
EMB = 128
H2 = 256
P = 4096
L = 50
E_TOT = P * L
TBL_PAD = 10240
HALF = 5120
NC = 2
NS = 16
NW = NC * NS


def _mm_body(e_ref, w_ref, o_ref):
    o_ref[...] = jnp.dot(e_ref[...], w_ref[...], preferred_element_type=jnp.float32)


def _proj_table(emb_pad, w_top):
    return pl.pallas_call(
        _mm_body,
        grid=(10,),
        in_specs=[
            pl.BlockSpec((1024, EMB), lambda i: (i, 0)),
            pl.BlockSpec((EMB, EMB), lambda i: (0, 0)),
        ],
        out_specs=pl.BlockSpec((1024, EMB), lambda i: (i, 0)),
        out_shape=jax.ShapeDtypeStruct((TBL_PAD, EMB), jnp.float32),
    )(emb_pad, w_top)


_EV_W = E_TOT // NW
_GCH = 128
_GN = _EV_W // _GCH


def _gather_rows(proj, flat_ids):
    mesh = plsc.VectorSubcoreMesh(core_axis_name="c", subcore_axis_name="s")

    @functools.partial(
        pl.kernel,
        mesh=mesh,
        out_type=jax.ShapeDtypeStruct((E_TOT, EMB), jnp.float32),
        scratch_types=[
            pltpu.VMEM((_EV_W,), jnp.int32),
            pltpu.VMEM((_GCH, EMB), jnp.float32),
            pltpu.SemaphoreType.DMA,
        ],
    )
    def k(proj_hbm, ids_hbm, out_hbm, idx_v, rows_v, sem):
        wid = lax.axis_index("s") * NC + lax.axis_index("c")
        base = wid * _EV_W
        pltpu.sync_copy(ids_hbm.at[pl.ds(base, _EV_W)], idx_v)

        def body(j, carry):
            pltpu.async_copy(
                proj_hbm.at[idx_v.at[pl.ds(j * _GCH, _GCH)]], rows_v, sem
            ).wait()
            pltpu.sync_copy(rows_v, out_hbm.at[pl.ds(base + j * _GCH, _GCH)])
            return carry

        lax.fori_loop(0, _GN, body, 0)

    return k(proj, flat_ids)


def _fuse_body(g_ref, f_ref, ids_ref, p_ref, wb_ref, be_ref, wi_ref, bi_ref,
               wda_ref, pe_ref, da_ref):
    z = g_ref[...] + jnp.dot(f_ref[...], wb_ref[...],
                             preferred_element_type=jnp.float32) + be_ref[...]
    z = jnp.maximum(z, 0.0)
    m = (ids_ref[...] != 0).astype(jnp.float32)
    zm = z * m
    pmat = p_ref[...]
    pooled = jnp.dot(pmat, zm, preferred_element_type=jnp.float32)
    lens = jnp.maximum(jnp.dot(pmat, m, preferred_element_type=jnp.float32), 1.0)
    pooled = pooled / lens
    pe = jnp.maximum(
        jnp.dot(pooled, wi_ref[...], preferred_element_type=jnp.float32)
        + bi_ref[...], 0.0)
    pe_ref[...] = pe
    da_ref[...] = jnp.dot(pe, wda_ref[...], preferred_element_type=jnp.float32)


def _fuse_pool(gathered, feats_flat, ids_col, pmat, w_bot, b_emb, w_inner,
               b_inner, w_da):
    blk = E_TOT // 32
    return pl.pallas_call(
        _fuse_body,
        grid=(32,),
        in_specs=[
            pl.BlockSpec((blk, EMB), lambda i: (i, 0)),
            pl.BlockSpec((blk, D_FEAT), lambda i: (i, 0)),
            pl.BlockSpec((blk, 1), lambda i: (i, 0)),
            pl.BlockSpec((P // 32, blk), lambda i: (0, 0)),
            pl.BlockSpec((D_FEAT, EMB), lambda i: (0, 0)),
            pl.BlockSpec((1, EMB), lambda i: (0, 0)),
            pl.BlockSpec((EMB, H2), lambda i: (0, 0)),
            pl.BlockSpec((1, H2), lambda i: (0, 0)),
            pl.BlockSpec((H2, 2), lambda i: (0, 0)),
        ],
        out_specs=[
            pl.BlockSpec((P // 32, H2), lambda i: (i, 0)),
            pl.BlockSpec((P // 32, 2), lambda i: (i, 0)),
        ],
        out_shape=[
            jax.ShapeDtypeStruct((P, H2), jnp.float32),
            jax.ShapeDtypeStruct((P, 2), jnp.float32),
        ],
    )(gathered, feats_flat, ids_col, pmat, w_bot, b_emb, w_inner, b_inner, w_da)


_OWN = (2 * HALF) // NW
_SCH = 2048
_SN = E_TOT // _SCH
_CAP = 9600
_RB = 64


def _scatter_edges(path_emb, flat_ids, p_of_e, zrows):
    mesh = plsc.VectorSubcoreMesh(core_axis_name="c", subcore_axis_name="s")

    @functools.partial(
        pl.kernel,
        mesh=mesh,
        out_type=jax.ShapeDtypeStruct((2 * HALF, H2), jnp.float32),
        compiler_params=pltpu.CompilerParams(needs_layout_passes=False),
        scratch_types=[
            pltpu.VMEM((_SCH,), jnp.int32),
            pltpu.VMEM((_SCH,), jnp.int32),
            pltpu.VMEM((_CAP + _RB + 16,), jnp.int32),
            pltpu.VMEM((_CAP + _RB + 16,), jnp.int32),
            pltpu.VMEM((_RB, H2), jnp.float32),
            pltpu.VMEM((_OWN + 1, H2), jnp.float32),
            pltpu.SemaphoreType.DMA,
        ],
    )
    def k(pe_hbm, ids_hbm, pofe_hbm, z_hbm, out_hbm, idv, pv, cl, cp,
          rows_v, acc, sem):
        wid = lax.axis_index("s") * NC + lax.axis_index("c")
        first_id = 1 + wid * _OWN

        pltpu.sync_copy(z_hbm, acc)

        def scan_chunk(kk, off):
            ebase = kk * _SCH
            pltpu.sync_copy(ids_hbm.at[pl.ds(ebase, _SCH)], idv)
            pltpu.sync_copy(pofe_hbm.at[pl.ds(ebase, _SCH)], pv)
            for g in range(_SCH // 16):
                x = idv[pl.ds(g * 16, 16)]
                p16 = pv[pl.ds(g * 16, 16)]
                offv = jnp.broadcast_to(off, (16,)).astype(jnp.int32)
                local = x - jnp.broadcast_to(first_id, (16,)).astype(jnp.int32)
                valid = jnp.logical_and(
                    jnp.logical_and(local >= jnp.zeros((16,), jnp.int32),
                                    local < jnp.full((16,), _OWN, jnp.int32)),
                    offv < jnp.full((16,), _CAP, jnp.int32))
                mi = valid.astype(jnp.int32)
                inc = plsc.cumsum(mi)
                pos = offv + inc - mi
                plsc.store_scatter(cl, [pos], local, mask=valid)
                plsc.store_scatter(cp, [pos], p16, mask=valid)
                off = off + inc[15]
            return off

        n = lax.fori_loop(0, _SN, scan_chunk, 0)

        for g in range(_RB // 16):
            cl[pl.ds(n + g * 16, 16)] = jnp.full((16,), _OWN, jnp.int32)
            cp[pl.ds(n + g * 16, 16)] = jnp.zeros((16,), jnp.int32)

        def accum_block(j, carry):
            pltpu.async_copy(pe_hbm.at[cp.at[pl.ds(j * _RB, _RB)]], rows_v,
                             sem).wait()

            base = j * _RB
            for gg in range(_RB // 16):
                vrow = cl[pl.ds(base + gg * 16, 16)]
                for r in range(16):
                    rowid = vrow[r]
                    rr = gg * 16 + r
                    for g in range(H2 // 16):
                        plsc.addupdate(acc.at[rowid, pl.ds(g * 16, 16)],
                                       rows_v[rr, pl.ds(g * 16, 16)])
            return carry

        nb = (n + _RB - 1) // _RB
        lax.fori_loop(0, nb, accum_block, 0)

        pltpu.sync_copy(acc.at[pl.ds(0, _OWN)],
                        out_hbm.at[pl.ds(wid * _OWN, _OWN)])

    return k(path_emb, flat_ids, p_of_e, zrows)


def _edge_body(acc_ref, ef_ref, wt_ref, wb_ref, bt_ref, w1_ref, b1_ref,
               w2_ref, b2_ref, p2e_ref, pred_ref):
    p2e = (jnp.dot(acc_ref[...], wt_ref[...], preferred_element_type=jnp.float32)
           + jnp.dot(ef_ref[...], wb_ref[...], preferred_element_type=jnp.float32)
           + bt_ref[...])
    p2e = jnp.maximum(p2e, 0.0)
    h = jnp.maximum(
        jnp.dot(p2e, w1_ref[...], preferred_element_type=jnp.float32)
        + b1_ref[...], 0.0)
    p2e_ref[...] = p2e
    pred_ref[...] = (jnp.dot(h, w2_ref[...], preferred_element_type=jnp.float32)
                     + b2_ref[...])


def _edge_mlps(edge_acc, edge_feats, w_it, w_ib, b_inter, w1, b1, w2, b2):
    blk = 2000
    return pl.pallas_call(
        _edge_body,
        grid=(5,),
        in_specs=[
            pl.BlockSpec((blk, H2), lambda i: (i, 0)),
            pl.BlockSpec((blk, D_FEAT), lambda i: (i, 0)),
            pl.BlockSpec((H2, H2), lambda i: (0, 0)),
            pl.BlockSpec((D_FEAT, H2), lambda i: (0, 0)),
            pl.BlockSpec((1, H2), lambda i: (0, 0)),
            pl.BlockSpec((H2, H2), lambda i: (0, 0)),
            pl.BlockSpec((1, H2), lambda i: (0, 0)),
            pl.BlockSpec((H2, EMB), lambda i: (0, 0)),
            pl.BlockSpec((1, EMB), lambda i: (0, 0)),
        ],
        out_specs=[
            pl.BlockSpec((blk, H2), lambda i: (i, 0)),
            pl.BlockSpec((blk, EMB), lambda i: (i, 0)),
        ],
        out_shape=[
            jax.ShapeDtypeStruct((EDGE_NUM, H2), jnp.float32),
            jax.ShapeDtypeStruct((EDGE_NUM, EMB), jnp.float32),
        ],
    )(edge_acc, edge_feats, w_it, w_ib, b_inter, w1, b1, w2, b2)


def kernel(path_segment_ids, path_segment_feats, edge_feats, emb_table, W_emb,
           b_emb, W_inner, b_inner, W_inter, b_inter, W_fwd1, b_fwd1, W_fwd2,
           b_fwd2, W_due, W_act):
    ids = path_segment_ids.astype(jnp.int32)
    flat_ids = ids.reshape(-1)
    feats_flat = path_segment_feats.reshape(E_TOT, D_FEAT)
    ids_col = flat_ids.reshape(E_TOT, 1)

    emb_pad = jnp.pad(emb_table, ((0, TBL_PAD - (EDGE_NUM + 1)), (0, 0)))
    w_top = W_emb[:EMB]
    w_bot = W_emb[EMB:]

    p_of_e = jnp.repeat(jnp.arange(P, dtype=jnp.int32), L)
    pmat = ((jnp.arange(E_TOT // 32, dtype=jnp.int32) // L)[None, :]
            == jnp.arange(P // 32, dtype=jnp.int32)[:, None]).astype(jnp.float32)
    zrows = jnp.zeros((_OWN + 1, H2), jnp.float32)

    proj = _proj_table(emb_pad, w_top)
    gathered = _gather_rows(proj, flat_ids)
    path_emb, da = _fuse_pool(
        gathered, feats_flat, ids_col, pmat, w_bot, b_emb.reshape(1, EMB),
        W_inner, b_inner.reshape(1, H2),
        jnp.concatenate([W_due, W_act], axis=1))

    edge_acc = _scatter_edges(path_emb, flat_ids, p_of_e, zrows)

    path2edge, pred = _edge_mlps(
        edge_acc[:EDGE_NUM], edge_feats, W_inter[:H2], W_inter[H2:],
        b_inter.reshape(1, H2), W_fwd1, b_fwd1.reshape(1, H2), W_fwd2,
        b_fwd2.reshape(1, EMB))

    return (pred, path2edge, da[:, 0], da[:, 1])


# speedup vs baseline: 1.7544x; 1.1440x over previous
import functools

import jax
import jax.numpy as jnp
from jax import lax
from jax.experimental import pallas as pl
from jax.experimental.pallas import tpu as pltpu
from jax.experimental.pallas import tpu_sc as plsc

EDGE_NUM = 10000
D_FEAT = 16
EMB = 128
H2 = 256
P = 4096
L = 50
E_TOT = P * L
TBL_PAD = 10240
HALF = 5120
NC = 2
NS = 16
NW = NC * NS


def _mm_body(e_ref, w_ref, o_ref):
    o_ref[...] = jnp.dot(e_ref[...], w_ref[...], preferred_element_type=jnp.float32)


def _proj_table(emb_pad, w_top):
    return pl.pallas_call(
        _mm_body,
        grid=(10,),
        in_specs=[
            pl.BlockSpec((1024, EMB), lambda i: (i, 0)),
            pl.BlockSpec((EMB, EMB), lambda i: (0, 0)),
        ],
        out_specs=pl.BlockSpec((1024, EMB), lambda i: (i, 0)),
        out_shape=jax.ShapeDtypeStruct((TBL_PAD, EMB), jnp.float32),
    )(emb_pad, w_top)


_EV_W = E_TOT // NW
_GCH = 128
_GN = _EV_W // _GCH


def _gather_rows(proj, flat_ids):
    mesh = plsc.VectorSubcoreMesh(core_axis_name="c", subcore_axis_name="s")

    @functools.partial(
        pl.kernel,
        mesh=mesh,
        out_type=jax.ShapeDtypeStruct((E_TOT, EMB), jnp.float32),
        scratch_types=[
            pltpu.VMEM((_EV_W,), jnp.int32),
            pltpu.VMEM((_GCH, EMB), jnp.float32),
            pltpu.VMEM((_GCH, EMB), jnp.float32),
            pltpu.SemaphoreType.DMA,
            pltpu.SemaphoreType.DMA,
        ],
    )
    def k(proj_hbm, ids_hbm, out_hbm, idx_v, rows0, rows1, sem0, sem1):
        wid = lax.axis_index("s") * NC + lax.axis_index("c")
        base = wid * _EV_W
        pltpu.sync_copy(ids_hbm.at[pl.ds(base, _EV_W)], idx_v)

        bufs = (rows0, rows1)
        sems = (sem0, sem1)
        pltpu.async_copy(proj_hbm.at[idx_v.at[pl.ds(0, _GCH)]], rows0, sem0)
        pltpu.async_copy(proj_hbm.at[idx_v.at[pl.ds(_GCH, _GCH)]], rows1, sem1)

        def body(jj, carry):
            for b in range(2):
                j = jj * 2 + b
                pltpu.make_async_copy(proj_hbm.at[pl.ds(0, _GCH)],
                                      bufs[b], sems[b]).wait()
                pltpu.sync_copy(bufs[b],
                                out_hbm.at[pl.ds(base + j * _GCH, _GCH)])

                @pl.when(j + 2 < _GN)
                def _():
                    pltpu.async_copy(
                        proj_hbm.at[idx_v.at[pl.ds((j + 2) * _GCH, _GCH)]],
                        bufs[b], sems[b])
            return carry

        lax.fori_loop(0, _GN // 2, body, 0)

    return k(proj, flat_ids)


def _fuse_body(g_ref, f_ref, ids_ref, p_ref, wb_ref, be_ref, wi_ref, bi_ref,
               wda_ref, pe_ref, da_ref):
    z = g_ref[...] + jnp.dot(f_ref[...], wb_ref[...],
                             preferred_element_type=jnp.float32) + be_ref[...]
    z = jnp.maximum(z, 0.0)
    m = (ids_ref[...] != 0).astype(jnp.float32)
    zm = z * m
    pmat = p_ref[...]
    pooled = jnp.dot(pmat, zm, preferred_element_type=jnp.float32)
    lens = jnp.maximum(jnp.dot(pmat, m, preferred_element_type=jnp.float32), 1.0)
    pooled = pooled / lens
    pe = jnp.maximum(
        jnp.dot(pooled, wi_ref[...], preferred_element_type=jnp.float32)
        + bi_ref[...], 0.0)
    pe_ref[...] = pe
    da_ref[...] = jnp.dot(pe, wda_ref[...], preferred_element_type=jnp.float32)


def _fuse_pool(gathered, feats_flat, ids_col, pmat, w_bot, b_emb, w_inner,
               b_inner, w_da):
    blk = E_TOT // 32
    return pl.pallas_call(
        _fuse_body,
        grid=(32,),
        in_specs=[
            pl.BlockSpec((blk, EMB), lambda i: (i, 0)),
            pl.BlockSpec((blk, D_FEAT), lambda i: (i, 0)),
            pl.BlockSpec((blk, 1), lambda i: (i, 0)),
            pl.BlockSpec((P // 32, blk), lambda i: (0, 0)),
            pl.BlockSpec((D_FEAT, EMB), lambda i: (0, 0)),
            pl.BlockSpec((1, EMB), lambda i: (0, 0)),
            pl.BlockSpec((EMB, H2), lambda i: (0, 0)),
            pl.BlockSpec((1, H2), lambda i: (0, 0)),
            pl.BlockSpec((H2, 2), lambda i: (0, 0)),
        ],
        out_specs=[
            pl.BlockSpec((P // 32, H2), lambda i: (i, 0)),
            pl.BlockSpec((P // 32, 2), lambda i: (i, 0)),
        ],
        out_shape=[
            jax.ShapeDtypeStruct((P, H2), jnp.float32),
            jax.ShapeDtypeStruct((P, 2), jnp.float32),
        ],
    )(gathered, feats_flat, ids_col, pmat, w_bot, b_emb, w_inner, b_inner, w_da)


_OWN = (2 * HALF) // NW
_SCH = 1024
_SN = E_TOT // _SCH
_CAP = 8192
_RB = 48
_CPK = _CAP + 2 * _RB + 16


def _scatter_edges(path_emb, flat_ids, zrows):
    mesh = plsc.VectorSubcoreMesh(core_axis_name="c", subcore_axis_name="s")

    @functools.partial(
        pl.kernel,
        mesh=mesh,
        out_type=jax.ShapeDtypeStruct((2 * HALF, H2), jnp.float32),
        compiler_params=pltpu.CompilerParams(needs_layout_passes=False),
        scratch_types=[
            pltpu.VMEM((_SCH,), jnp.int32),
            pltpu.VMEM((_SCH,), jnp.int32),
            pltpu.VMEM((_CPK,), jnp.int32),
            pltpu.VMEM((_RB,), jnp.int32),
            pltpu.VMEM((_RB,), jnp.int32),
            pltpu.VMEM((_RB, H2), jnp.float32),
            pltpu.VMEM((_RB, H2), jnp.float32),
            pltpu.VMEM((_OWN + 1, H2), jnp.float32),
            pltpu.SemaphoreType.DMA,
            pltpu.SemaphoreType.DMA,
            pltpu.SemaphoreType.DMA,
            pltpu.SemaphoreType.DMA,
        ],
    )
    def k(pe_hbm, ids_hbm, z_hbm, out_hbm, idv0, idv1, cpk, ixp0, ixp1,
          rows0, rows1, acc, sa0, sa1, sb0, sb1):
        wid = lax.axis_index("s") * NC + lax.axis_index("c")
        first_id = 1 + wid * _OWN
        iota_f = lax.iota(jnp.int32, 16).astype(jnp.float32)
        idbufs, isems = (idv0, idv1), (sa0, sa1)
        ixbufs, rbufs, rsems = (ixp0, ixp1), (rows0, rows1), (sb0, sb1)

        pltpu.sync_copy(z_hbm, acc)

        pltpu.async_copy(ids_hbm.at[pl.ds(0, _SCH)], idv0, sa0)
        pltpu.async_copy(ids_hbm.at[pl.ds(_SCH, _SCH)], idv1, sa1)

        def scan_pair(kk, off):
            for b in range(2):
                ck = kk * 2 + b
                buf = idbufs[b]
                pltpu.make_async_copy(ids_hbm.at[pl.ds(0, _SCH)], buf,
                                      isems[b]).wait()
                ebase = ck * _SCH
                for g in range(_SCH // 16):
                    x = buf[pl.ds(g * 16, 16)]
                    local = x - jnp.broadcast_to(first_id, (16,)).astype(jnp.int32)
                    ef = (jnp.broadcast_to(
                        (ebase + g * 16).astype(jnp.float32) + 0.5, (16,))
                        + iota_f)
                    p16 = (ef * jnp.float32(0.02)).astype(jnp.int32)
                    word = jnp.bitwise_or(local, p16 << 9)
                    offv = jnp.broadcast_to(off, (16,)).astype(jnp.int32)
                    valid = jnp.logical_and(
                        jnp.logical_and(local >= jnp.zeros((16,), jnp.int32),
                                        local < jnp.full((16,), _OWN, jnp.int32)),
                        offv < jnp.full((16,), _CAP, jnp.int32))
                    mi = valid.astype(jnp.int32)
                    inc = plsc.cumsum(mi)
                    plsc.store_scatter(cpk, [offv + inc - mi], word, mask=valid)
                    off = off + inc[15]

                @pl.when(ck + 2 < _SN)
                def _():
                    pltpu.async_copy(ids_hbm.at[pl.ds((ck + 2) * _SCH, _SCH)],
                                     buf, isems[b])
            return off

        n = lax.fori_loop(0, _SN // 2, scan_pair, 0)

        for g in range(2 * _RB // 16):
            cpk[pl.ds(n + g * 16, 16)] = jnp.full((16,), _OWN, jnp.int32)

        nbe2 = jnp.maximum(((n + _RB - 1) // _RB + 1) // 2, 1)

        def issue(blk, b):
            for gg in range(_RB // 16):
                w = cpk[pl.ds(blk * _RB + gg * 16, 16)]
                ixbufs[b][pl.ds(gg * 16, 16)] = lax.shift_right_logical(w, 9)
            pltpu.async_copy(pe_hbm.at[ixbufs[b]], rbufs[b], rsems[b])

        issue(0, 0)
        issue(1, 1)

        def acc_pair(jj, carry):
            for b in range(2):
                j = jj * 2 + b
                pltpu.make_async_copy(pe_hbm.at[pl.ds(0, _RB)], rbufs[b],
                                      rsems[b]).wait()
                base = j * _RB
                for gg in range(_RB // 16):
                    vrow = cpk[pl.ds(base + gg * 16, 16)]
                    for r in range(16):
                        rowid = jnp.bitwise_and(vrow[r], 511)
                        rr = gg * 16 + r
                        for g in range(H2 // 16):
                            plsc.addupdate(acc.at[rowid, pl.ds(g * 16, 16)],
                                           rbufs[b][rr, pl.ds(g * 16, 16)])

                @pl.when(j + 2 < 2 * nbe2)
                def _():
                    issue(j + 2, b)
            return carry

        lax.fori_loop(0, nbe2, acc_pair, 0)

        pltpu.sync_copy(acc.at[pl.ds(0, _OWN)],
                        out_hbm.at[pl.ds(wid * _OWN, _OWN)])

    return k(path_emb, flat_ids, zrows)


def _edge_body(acc_ref, ef_ref, wt_ref, wb_ref, bt_ref, w1_ref, b1_ref,
               w2_ref, b2_ref, p2e_ref, pred_ref):
    p2e = (jnp.dot(acc_ref[...], wt_ref[...], preferred_element_type=jnp.float32)
           + jnp.dot(ef_ref[...], wb_ref[...], preferred_element_type=jnp.float32)
           + bt_ref[...])
    p2e = jnp.maximum(p2e, 0.0)
    h = jnp.maximum(
        jnp.dot(p2e, w1_ref[...], preferred_element_type=jnp.float32)
        + b1_ref[...], 0.0)
    p2e_ref[...] = p2e
    pred_ref[...] = (jnp.dot(h, w2_ref[...], preferred_element_type=jnp.float32)
                     + b2_ref[...])


def _edge_mlps(edge_acc, edge_feats, w_it, w_ib, b_inter, w1, b1, w2, b2):
    blk = 2000
    return pl.pallas_call(
        _edge_body,
        grid=(5,),
        in_specs=[
            pl.BlockSpec((blk, H2), lambda i: (i, 0)),
            pl.BlockSpec((blk, D_FEAT), lambda i: (i, 0)),
            pl.BlockSpec((H2, H2), lambda i: (0, 0)),
            pl.BlockSpec((D_FEAT, H2), lambda i: (0, 0)),
            pl.BlockSpec((1, H2), lambda i: (0, 0)),
            pl.BlockSpec((H2, H2), lambda i: (0, 0)),
            pl.BlockSpec((1, H2), lambda i: (0, 0)),
            pl.BlockSpec((H2, EMB), lambda i: (0, 0)),
            pl.BlockSpec((1, EMB), lambda i: (0, 0)),
        ],
        out_specs=[
            pl.BlockSpec((blk, H2), lambda i: (i, 0)),
            pl.BlockSpec((blk, EMB), lambda i: (i, 0)),
        ],
        out_shape=[
            jax.ShapeDtypeStruct((EDGE_NUM, H2), jnp.float32),
            jax.ShapeDtypeStruct((EDGE_NUM, EMB), jnp.float32),
        ],
    )(edge_acc, edge_feats, w_it, w_ib, b_inter, w1, b1, w2, b2)


def kernel(path_segment_ids, path_segment_feats, edge_feats, emb_table, W_emb,
           b_emb, W_inner, b_inner, W_inter, b_inter, W_fwd1, b_fwd1, W_fwd2,
           b_fwd2, W_due, W_act):
    ids = path_segment_ids.astype(jnp.int32)
    flat_ids = ids.reshape(-1)
    feats_flat = path_segment_feats.reshape(E_TOT, D_FEAT)
    ids_col = flat_ids.reshape(E_TOT, 1)

    emb_pad = jnp.pad(emb_table, ((0, TBL_PAD - (EDGE_NUM + 1)), (0, 0)))
    w_top = W_emb[:EMB]
    w_bot = W_emb[EMB:]

    pmat = ((jnp.arange(E_TOT // 32, dtype=jnp.int32) // L)[None, :]
            == jnp.arange(P // 32, dtype=jnp.int32)[:, None]).astype(jnp.float32)
    zrows = jnp.zeros((_OWN + 1, H2), jnp.float32)

    proj = _proj_table(emb_pad, w_top)
    gathered = _gather_rows(proj, flat_ids)
    path_emb, da = _fuse_pool(
        gathered, feats_flat, ids_col, pmat, w_bot, b_emb.reshape(1, EMB),
        W_inner, b_inner.reshape(1, H2),
        jnp.concatenate([W_due, W_act], axis=1))

    edge_acc = _scatter_edges(path_emb, flat_ids, zrows)

    path2edge, pred = _edge_mlps(
        edge_acc[:EDGE_NUM], edge_feats, W_inter[:H2], W_inter[H2:],
        b_inter.reshape(1, H2), W_fwd1, b_fwd1.reshape(1, H2), W_fwd2,
        b_fwd2.reshape(1, EMB))

    return (pred, path2edge, da[:, 0], da[:, 1])
